# baseline (speedup 1.0000x reference)
import jax
import jax.numpy as jnp
from jax.experimental import pallas as pl

N = 10000
NTYPES = 2


def _combine_body(a_ref, b_ref, c_ref, d_ref, o_ref):
    o_ref[...] = (a_ref[...] + b_ref[...] + c_ref[...] + d_ref[...]) / 4.0


def _att_pool(h, node_type, gw, gb):
    gate = (h @ gw + gb)[:, 0]
    gmax = jax.ops.segment_max(gate, node_type, num_segments=NTYPES)
    ge = jnp.exp(gate - gmax[node_type])
    gs = jax.ops.segment_sum(ge, node_type, num_segments=NTYPES)
    attn = ge / gs[node_type]
    return jax.ops.segment_sum(h * attn[:, None], node_type, num_segments=NTYPES)


def kernel(x, edge_index, node_type, W0, b0, W1, b1, gateW0, gateb0, gateW1, gateb1, Wp00, bp00, Wp01, bp01, Wp10, bp10, Wp11, bp11):
    loops = jnp.arange(N)
    src = jnp.concatenate([edge_index[0], loops])
    dst = jnp.concatenate([edge_index[1], loops])
    deg_out = jnp.maximum(jnp.bincount(src, length=N), 1).astype(jnp.float32)
    deg_in = jnp.maximum(jnp.bincount(dst, length=N), 1).astype(jnp.float32)
    norm_src = jax.lax.rsqrt(deg_out)
    norm_dst = jax.lax.rsqrt(deg_in)
    h = x
    p0 = _att_pool(h, node_type, gateW0, gateb0)
    o00 = p0[0] @ Wp00 + bp00
    o01 = p0[1] @ Wp10 + bp10
    m = (h * norm_src[:, None])[src]
    agg = jax.ops.segment_sum(m, dst, num_segments=N)
    h = jax.nn.relu((agg * norm_dst[:, None]) @ W0 + b0)
    p1 = _att_pool(h, node_type, gateW1, gateb1)
    o10 = p1[0] @ Wp01 + bp01
    o11 = p1[1] @ Wp11 + bp11
    hg = pl.pallas_call(
        _combine_body,
        out_shape=jax.ShapeDtypeStruct((1, 64), jnp.float32),
    )(o00.reshape(1, 64), o01.reshape(1, 64), o10.reshape(1, 64), o11.reshape(1, 64))
    return hg.reshape(64)


# speedup vs baseline: 3.4516x; 3.4516x over previous
import functools

import jax
import jax.numpy as jnp
from jax import lax
from jax.experimental import pallas as pl
from jax.experimental.pallas import tpu as pltpu
from jax.experimental.pallas import tpu_sc as plsc

N = 10000
N_PAD = 10240
E = 160000
IN_DIM = 256
HID = 512
OUT_DIM = 64
HALF = 128

NTILES = 16
CHUNK = 128
E_PAD = 163840
EPT = E_PAD // NTILES
NFULL = EPT // CHUNK
RPT = N_PAD // NTILES
GROUPS = CHUNK // 16

BLK = 1024
NBLK = N_PAD // BLK

@functools.lru_cache(maxsize=None)
def _sc_mesh():
    return plsc.VectorSubcoreMesh(core_axis_name="c", subcore_axis_name="s")


def _bincount_body(src_hbm, dst_hbm, cs_hbm, cd_hbm,
                   idx_v, ones_v, cnt_sh):
    c = lax.axis_index("c")
    s = lax.axis_index("s")

    ones16 = jnp.full((16,), 1.0, jnp.float32)

    def fill(i, _):
        def fcol(g, _):
            ones_v[i, pl.ds(g * 16, 16)] = ones16
            return 0
        lax.fori_loop(0, HALF // 16, fcol, 0)
        return 0
    lax.fori_loop(0, CHUNK, fill, 0)

    def run(e_hbm, out_hbm):
        def seed_body(k, _):
            pltpu.sync_copy(ones_v, cnt_sh.at[pl.ds(s * RPT + k * CHUNK, CHUNK)])
            return 0
        lax.fori_loop(0, RPT // CHUNK, seed_body, 0)
        plsc.subcore_barrier()

        base = s * EPT

        def chunk_body(j, _):
            pltpu.sync_copy(e_hbm.at[pl.ds(base + j * CHUNK, CHUNK)], idx_v)
            pltpu.sync_copy(ones_v, cnt_sh.at[idx_v], add=True)
            return 0
        lax.fori_loop(0, NFULL, chunk_body, 0)

        plsc.subcore_barrier()

        def out_body(k, _):
            r0 = s * RPT + k * CHUNK
            pltpu.sync_copy(cnt_sh.at[pl.ds(r0, CHUNK)], ones_v)
            pltpu.sync_copy(ones_v, out_hbm.at[pl.ds(r0, CHUNK)])
            return 0
        lax.fori_loop(0, RPT // CHUNK, out_body, 0)

    @pl.when(c == 0)
    def _():
        run(src_hbm, cs_hbm)

    @pl.when(c == 1)
    def _():
        run(dst_hbm, cd_hbm)


@functools.lru_cache(maxsize=None)
def _bincount():
    return functools.partial(
        pl.kernel,
        out_type=[
            jax.ShapeDtypeStruct((N_PAD, HALF), jnp.float32),
            jax.ShapeDtypeStruct((N_PAD, HALF), jnp.float32),
        ],
        mesh=_sc_mesh(),
        scratch_types=[
            pltpu.VMEM((CHUNK,), jnp.int32),
            pltpu.VMEM((CHUNK, HALF), jnp.float32),
            pltpu.VMEM_SHARED((N_PAD, HALF), jnp.float32),
        ],
    )(_bincount_body)


def _agg_body(xsl_hbm, xsr_hbm, src_hbm, dst_hbm, al_hbm, ar_hbm,
              gi_v, di_v, rows_v, acc_sh, sem):
    c = lax.axis_index("c")
    s = lax.axis_index("s")

    def run(xs_hbm, out_hbm):
        def seed_body(k, _):
            r0 = s * RPT + k * CHUNK
            pltpu.sync_copy(xs_hbm.at[pl.ds(r0, CHUNK)], rows_v)
            pltpu.sync_copy(rows_v, acc_sh.at[pl.ds(r0, CHUNK)])
            return 0
        lax.fori_loop(0, RPT // CHUNK, seed_body, 0)
        plsc.subcore_barrier()

        base = s * EPT

        def chunk_body(j, _):
            pltpu.sync_copy(src_hbm.at[pl.ds(base + j * CHUNK, CHUNK)], gi_v)
            pltpu.sync_copy(dst_hbm.at[pl.ds(base + j * CHUNK, CHUNK)], di_v)
            pltpu.async_copy(xs_hbm.at[gi_v], rows_v, sem).wait()
            pltpu.sync_copy(rows_v, acc_sh.at[di_v], add=True)
            return 0
        lax.fori_loop(0, NFULL, chunk_body, 0)

        plsc.subcore_barrier()

        def out_body(k, _):
            r0 = s * RPT + k * CHUNK
            pltpu.sync_copy(acc_sh.at[pl.ds(r0, CHUNK)], rows_v)
            pltpu.sync_copy(rows_v, out_hbm.at[pl.ds(r0, CHUNK)])
            return 0
        lax.fori_loop(0, RPT // CHUNK, out_body, 0)

    @pl.when(c == 0)
    def _():
        run(xsl_hbm, al_hbm)

    @pl.when(c == 1)
    def _():
        run(xsr_hbm, ar_hbm)


@functools.lru_cache(maxsize=None)
def _aggregate():
    return functools.partial(
        pl.kernel,
        out_type=[
            jax.ShapeDtypeStruct((N_PAD, HALF), jnp.float32),
            jax.ShapeDtypeStruct((N_PAD, HALF), jnp.float32),
        ],
        mesh=_sc_mesh(),
        scratch_types=[
            pltpu.VMEM((CHUNK,), jnp.int32),
            pltpu.VMEM((CHUNK,), jnp.int32),
            pltpu.VMEM((CHUNK, HALF), jnp.float32),
            pltpu.VMEM_SHARED((N_PAD, HALF), jnp.float32),
            pltpu.SemaphoreType.DMA,
        ],
    )(_agg_body)


def _online_pool_update(i, g, feats, nt_col, acc_ref, m_ref, s_ref):
    @pl.when(i == 0)
    def _():
        m_ref[0] = -jnp.inf
        m_ref[1] = -jnp.inf
        s_ref[0] = 0.0
        s_ref[1] = 0.0
        acc_ref[...] = jnp.zeros_like(acc_ref)

    for t in range(2):
        mask = nt_col == t
        gm = jnp.max(jnp.where(mask, g, -jnp.inf))
        m_old = m_ref[t]
        m_new = jnp.maximum(m_old, gm)
        scale = jnp.where(m_old == -jnp.inf, 0.0, jnp.exp(m_old - m_new))
        w = jnp.where(mask, jnp.exp(g - m_new), 0.0)
        s_ref[t] = s_ref[t] * scale + jnp.sum(w)
        contrib = lax.dot_general(
            w, feats, (((0,), (0,)), ((), ())),
            preferred_element_type=jnp.float32)
        acc_ref[t:t + 1, :] = acc_ref[t:t + 1, :] * scale + contrib
        m_ref[t] = m_new


def _pool_finish(acc_ref, s_ref):
    s0 = s_ref[0]
    s1 = s_ref[1]
    p0 = jnp.where(s0 > 0, acc_ref[0:1, :] / s0, 0.0)
    p1 = jnp.where(s1 > 0, acc_ref[1:2, :] / s1, 0.0)
    return jnp.concatenate([p0, p1], axis=1)


def _tc1_body(x_ref, nt_ref, cs_ref, gw_ref, wq_ref,
              xsl_ref, xsr_ref, o0_ref, acc_ref, m_ref, s_ref):
    i = pl.program_id(0)
    xb = x_ref[...]
    nsrc = lax.rsqrt(cs_ref[:, 0:1])
    xs = xb * nsrc
    xsl_ref[...] = xs[:, :HALF]
    xsr_ref[...] = xs[:, HALF:]

    g8 = jnp.dot(xb, gw_ref[...], preferred_element_type=jnp.float32)
    _online_pool_update(i, g8[:, 0:1], xb, nt_ref[:, 0:1],
                        acc_ref, m_ref, s_ref)

    @pl.when(i == pl.num_programs(0) - 1)
    def _():
        pcat = _pool_finish(acc_ref, s_ref)
        o0_ref[...] = jnp.dot(pcat, wq_ref[...],
                              preferred_element_type=jnp.float32)


def _tc1(x, nt16, cs16, gw08, Wq0):
    return pl.pallas_call(
        _tc1_body,
        grid=(NBLK,),
        in_specs=[
            pl.BlockSpec((BLK, IN_DIM), lambda i: (i, 0)),
            pl.BlockSpec((BLK, 16), lambda i: (i, 0)),
            pl.BlockSpec((BLK, HALF), lambda i: (i, 0)),
            pl.BlockSpec((IN_DIM, 8), lambda i: (0, 0)),
            pl.BlockSpec((2 * IN_DIM, OUT_DIM), lambda i: (0, 0)),
        ],
        out_specs=[
            pl.BlockSpec((BLK, HALF), lambda i: (i, 0)),
            pl.BlockSpec((BLK, HALF), lambda i: (i, 0)),
            pl.BlockSpec((1, OUT_DIM), lambda i: (0, 0)),
        ],
        out_shape=[
            jax.ShapeDtypeStruct((N_PAD, HALF), jnp.float32),
            jax.ShapeDtypeStruct((N_PAD, HALF), jnp.float32),
            jax.ShapeDtypeStruct((1, OUT_DIM), jnp.float32),
        ],
        scratch_shapes=[
            pltpu.VMEM((2, IN_DIM), jnp.float32),
            pltpu.SMEM((2,), jnp.float32),
            pltpu.SMEM((2,), jnp.float32),
        ],
    )(x, nt16, cs16, gw08, Wq0)


def _tc2_body(al_ref, ar_ref, nt_ref, cd_ref, w0_ref, b0_ref, gw_ref,
              wq_ref, o0_ref, bsum_ref, out_ref, acc_ref, m_ref, s_ref):
    i = pl.program_id(0)
    nd = lax.rsqrt(cd_ref[:, 0:1])
    al = al_ref[...] * nd
    ar = ar_ref[...] * nd
    z = (jnp.dot(al, w0_ref[:HALF, :], preferred_element_type=jnp.float32)
         + jnp.dot(ar, w0_ref[HALF:, :], preferred_element_type=jnp.float32))
    h = jnp.maximum(z + b0_ref[...], 0.0)

    g8 = jnp.dot(h, gw_ref[...], preferred_element_type=jnp.float32)
    _online_pool_update(i, g8[:, 0:1], h, nt_ref[:, 0:1],
                        acc_ref, m_ref, s_ref)

    @pl.when(i == pl.num_programs(0) - 1)
    def _():
        pcat = _pool_finish(acc_ref, s_ref)
        o1 = jnp.dot(pcat, wq_ref[...], preferred_element_type=jnp.float32)
        out_ref[...] = (o0_ref[...] + o1 + bsum_ref[...]) * 0.25


def _tc2(aggl, aggr, nt16, cd16, W0, b0r, gw18, Wq1, o0, bsum):
    return pl.pallas_call(
        _tc2_body,
        grid=(NBLK,),
        in_specs=[
            pl.BlockSpec((BLK, HALF), lambda i: (i, 0)),
            pl.BlockSpec((BLK, HALF), lambda i: (i, 0)),
            pl.BlockSpec((BLK, 16), lambda i: (i, 0)),
            pl.BlockSpec((BLK, HALF), lambda i: (i, 0)),
            pl.BlockSpec((IN_DIM, HID), lambda i: (0, 0)),
            pl.BlockSpec((1, HID), lambda i: (0, 0)),
            pl.BlockSpec((HID, 8), lambda i: (0, 0)),
            pl.BlockSpec((2 * HID, OUT_DIM), lambda i: (0, 0)),
            pl.BlockSpec((1, OUT_DIM), lambda i: (0, 0)),
            pl.BlockSpec((1, OUT_DIM), lambda i: (0, 0)),
        ],
        out_specs=pl.BlockSpec((1, OUT_DIM), lambda i: (0, 0)),
        out_shape=jax.ShapeDtypeStruct((1, OUT_DIM), jnp.float32),
        scratch_shapes=[
            pltpu.VMEM((2, HID), jnp.float32),
            pltpu.SMEM((2,), jnp.float32),
            pltpu.SMEM((2,), jnp.float32),
        ],
    )(aggl, aggr, nt16, cd16, W0, b0r, gw18, Wq1, o0, bsum)


def kernel(x, edge_index, node_type, W0, b0, W1, b1, gateW0, gateb0,
           gateW1, gateb1, Wp00, bp00, Wp01, bp01, Wp10, bp10, Wp11, bp11):
    epad = E_PAD - E
    src = jnp.pad(edge_index[0].astype(jnp.int32), (0, epad),
                  constant_values=N_PAD - 1)
    dst = jnp.pad(edge_index[1].astype(jnp.int32), (0, epad),
                  constant_values=N_PAD - 1)
    pad = N_PAD - N
    xp = jnp.pad(x, ((0, pad), (0, 0)))
    ntp = jnp.pad(node_type.astype(jnp.int32), (0, pad), constant_values=2)
    nt16 = jnp.broadcast_to(ntp[:, None], (N_PAD, 16))

    cs16, cd16 = _bincount()(src, dst)

    gw08 = jnp.broadcast_to(gateW0, (IN_DIM, 8))
    Wq0 = jnp.concatenate([Wp00, Wp10], axis=0)
    xsl, xsr, o0 = _tc1(xp, nt16, cs16, gw08, Wq0)

    aggl, aggr = _aggregate()(xsl, xsr, src, dst)

    gw18 = jnp.broadcast_to(gateW1, (HID, 8))
    Wq1 = jnp.concatenate([Wp01, Wp11], axis=0)
    bsum = (bp00 + bp01 + bp10 + bp11).reshape(1, OUT_DIM)
    out = _tc2(aggl, aggr, nt16, cd16, W0, b0.reshape(1, HID), gw18, Wq1,
               o0, bsum)
    return out.reshape(OUT_DIM)


# speedup vs baseline: 4.2889x; 1.2426x over previous
import functools

import jax
import jax.numpy as jnp
from jax import lax
from jax.experimental import pallas as pl
from jax.experimental.pallas import tpu as pltpu
from jax.experimental.pallas import tpu_sc as plsc

N = 10000
N_PAD = 10240
E = 160000
IN_DIM = 256
HID = 512
OUT_DIM = 64
HALF = 128

NTILES = 16
CHUNK = 128
E_PAD = 163840
EPT = E_PAD // NTILES
NFULL = EPT // CHUNK
RPT = N_PAD // NTILES
GROUPS = CHUNK // 16

BLK = 1024
NBLK = N_PAD // BLK

@functools.lru_cache(maxsize=None)
def _sc_mesh():
    return plsc.VectorSubcoreMesh(core_axis_name="c", subcore_axis_name="s")


def _bincount_body(src_hbm, dst_hbm, cs_hbm, cd_hbm,
                   idx_v, ones_v, cnt_sh):
    c = lax.axis_index("c")
    s = lax.axis_index("s")

    ones16 = jnp.full((16,), 1.0, jnp.float32)

    def fill(i, _):
        def fcol(g, _):
            ones_v[i, pl.ds(g * 16, 16)] = ones16
            return 0
        lax.fori_loop(0, HALF // 16, fcol, 0)
        return 0
    lax.fori_loop(0, CHUNK, fill, 0)

    def run(e_hbm, out_hbm):
        def seed_body(k, _):
            pltpu.sync_copy(ones_v, cnt_sh.at[pl.ds(s * RPT + k * CHUNK, CHUNK)])
            return 0
        lax.fori_loop(0, RPT // CHUNK, seed_body, 0)
        plsc.subcore_barrier()

        base = s * EPT

        def chunk_body(j, _):
            pltpu.sync_copy(e_hbm.at[pl.ds(base + j * CHUNK, CHUNK)], idx_v)
            pltpu.sync_copy(ones_v, cnt_sh.at[idx_v], add=True)
            return 0
        lax.fori_loop(0, NFULL, chunk_body, 0)

        plsc.subcore_barrier()

        def out_body(k, _):
            r0 = s * RPT + k * CHUNK
            pltpu.sync_copy(cnt_sh.at[pl.ds(r0, CHUNK)], ones_v)
            pltpu.sync_copy(ones_v, out_hbm.at[pl.ds(r0, CHUNK)])
            return 0
        lax.fori_loop(0, RPT // CHUNK, out_body, 0)

    @pl.when(c == 0)
    def _():
        run(src_hbm, cs_hbm)

    @pl.when(c == 1)
    def _():
        run(dst_hbm, cd_hbm)


@functools.lru_cache(maxsize=None)
def _bincount():
    return functools.partial(
        pl.kernel,
        out_type=[
            jax.ShapeDtypeStruct((N_PAD, HALF), jnp.float32),
            jax.ShapeDtypeStruct((N_PAD, HALF), jnp.float32),
        ],
        mesh=_sc_mesh(),
        scratch_types=[
            pltpu.VMEM((CHUNK,), jnp.int32),
            pltpu.VMEM((CHUNK, HALF), jnp.float32),
            pltpu.VMEM_SHARED((N_PAD, HALF), jnp.float32),
        ],
    )(_bincount_body)


def _agg_body(xsl_hbm, xsr_hbm, src_hbm, dst_hbm, al_hbm, ar_hbm,
              gi0_v, di0_v, rows0_v, gi1_v, di1_v, rows1_v, acc_sh,
              sem0, sem1):
    c = lax.axis_index("c")
    s = lax.axis_index("s")

    def run(xs_hbm, out_hbm):
        def seed_body(k, _):
            r0 = s * RPT + k * CHUNK
            pltpu.sync_copy(xs_hbm.at[pl.ds(r0, CHUNK)], rows0_v)
            pltpu.sync_copy(rows0_v, acc_sh.at[pl.ds(r0, CHUNK)])
            return 0
        lax.fori_loop(0, RPT // CHUNK, seed_body, 0)
        plsc.subcore_barrier()

        base = s * EPT
        bufs = ((gi0_v, di0_v, rows0_v, sem0), (gi1_v, di1_v, rows1_v, sem1))

        def load_idx(j, gi, di):
            pltpu.sync_copy(src_hbm.at[pl.ds(base + j * CHUNK, CHUNK)], gi)
            pltpu.sync_copy(dst_hbm.at[pl.ds(base + j * CHUNK, CHUNK)], di)

        for b in range(2):
            gi, di, rows, sem = bufs[b]
            load_idx(b, gi, di)
            pltpu.async_copy(xs_hbm.at[gi], rows, sem)

        def chunk_body(j2, _):
            for b in range(2):
                jj = j2 * 2 + b
                gi, di, rows, sem = bufs[b]
                pltpu.make_async_copy(xs_hbm.at[gi], rows, sem).wait()
                pltpu.sync_copy(rows, acc_sh.at[di], add=True)

                @pl.when(jj + 2 < NFULL)
                def _():
                    load_idx(jj + 2, gi, di)
                    pltpu.async_copy(xs_hbm.at[gi], rows, sem)
            return 0
        lax.fori_loop(0, NFULL // 2, chunk_body, 0)

        plsc.subcore_barrier()

        def out_body(k, _):
            r0 = s * RPT + k * CHUNK
            pltpu.sync_copy(acc_sh.at[pl.ds(r0, CHUNK)], rows0_v)
            pltpu.sync_copy(rows0_v, out_hbm.at[pl.ds(r0, CHUNK)])
            return 0
        lax.fori_loop(0, RPT // CHUNK, out_body, 0)

    @pl.when(c == 0)
    def _():
        run(xsl_hbm, al_hbm)

    @pl.when(c == 1)
    def _():
        run(xsr_hbm, ar_hbm)


@functools.lru_cache(maxsize=None)
def _aggregate():
    return functools.partial(
        pl.kernel,
        out_type=[
            jax.ShapeDtypeStruct((N_PAD, HALF), jnp.float32),
            jax.ShapeDtypeStruct((N_PAD, HALF), jnp.float32),
        ],
        mesh=_sc_mesh(),
        scratch_types=[
            pltpu.VMEM((CHUNK,), jnp.int32),
            pltpu.VMEM((CHUNK,), jnp.int32),
            pltpu.VMEM((CHUNK, HALF), jnp.float32),
            pltpu.VMEM((CHUNK,), jnp.int32),
            pltpu.VMEM((CHUNK,), jnp.int32),
            pltpu.VMEM((CHUNK, HALF), jnp.float32),
            pltpu.VMEM_SHARED((N_PAD, HALF), jnp.float32),
            pltpu.SemaphoreType.DMA,
            pltpu.SemaphoreType.DMA,
        ],
    )(_agg_body)


def _online_pool_update(i, g, feats, nt_col, acc_ref, m_ref, s_ref):
    @pl.when(i == 0)
    def _():
        m_ref[0] = -jnp.inf
        m_ref[1] = -jnp.inf
        s_ref[0] = 0.0
        s_ref[1] = 0.0
        acc_ref[...] = jnp.zeros_like(acc_ref)

    for t in range(2):
        mask = nt_col == t
        gm = jnp.max(jnp.where(mask, g, -jnp.inf))
        m_old = m_ref[t]
        m_new = jnp.maximum(m_old, gm)
        scale = jnp.where(m_old == -jnp.inf, 0.0, jnp.exp(m_old - m_new))
        w = jnp.where(mask, jnp.exp(g - m_new), 0.0)
        s_ref[t] = s_ref[t] * scale + jnp.sum(w)
        contrib = lax.dot_general(
            w, feats, (((0,), (0,)), ((), ())),
            preferred_element_type=jnp.float32)
        acc_ref[t:t + 1, :] = acc_ref[t:t + 1, :] * scale + contrib
        m_ref[t] = m_new


def _pool_finish(acc_ref, s_ref):
    s0 = s_ref[0]
    s1 = s_ref[1]
    p0 = jnp.where(s0 > 0, acc_ref[0:1, :] / s0, 0.0)
    p1 = jnp.where(s1 > 0, acc_ref[1:2, :] / s1, 0.0)
    return jnp.concatenate([p0, p1], axis=1)


def _tc1_body(x_ref, nt_ref, cs_ref, gw_ref, wq_ref,
              xsl_ref, xsr_ref, o0_ref, acc_ref, m_ref, s_ref):
    i = pl.program_id(0)
    xb = x_ref[...]
    nsrc = lax.rsqrt(cs_ref[:, 0:1])
    xs = xb * nsrc
    xsl_ref[...] = xs[:, :HALF]
    xsr_ref[...] = xs[:, HALF:]

    g8 = jnp.dot(xb, gw_ref[...], preferred_element_type=jnp.float32)
    _online_pool_update(i, g8[:, 0:1], xb, nt_ref[:, 0:1],
                        acc_ref, m_ref, s_ref)

    @pl.when(i == pl.num_programs(0) - 1)
    def _():
        pcat = _pool_finish(acc_ref, s_ref)
        o0_ref[...] = jnp.dot(pcat, wq_ref[...],
                              preferred_element_type=jnp.float32)


def _tc1(x, nt16, cs16, gw08, Wq0):
    return pl.pallas_call(
        _tc1_body,
        grid=(NBLK,),
        in_specs=[
            pl.BlockSpec((BLK, IN_DIM), lambda i: (i, 0)),
            pl.BlockSpec((BLK, 16), lambda i: (i, 0)),
            pl.BlockSpec((BLK, HALF), lambda i: (i, 0)),
            pl.BlockSpec((IN_DIM, 8), lambda i: (0, 0)),
            pl.BlockSpec((2 * IN_DIM, OUT_DIM), lambda i: (0, 0)),
        ],
        out_specs=[
            pl.BlockSpec((BLK, HALF), lambda i: (i, 0)),
            pl.BlockSpec((BLK, HALF), lambda i: (i, 0)),
            pl.BlockSpec((1, OUT_DIM), lambda i: (0, 0)),
        ],
        out_shape=[
            jax.ShapeDtypeStruct((N_PAD, HALF), jnp.float32),
            jax.ShapeDtypeStruct((N_PAD, HALF), jnp.float32),
            jax.ShapeDtypeStruct((1, OUT_DIM), jnp.float32),
        ],
        scratch_shapes=[
            pltpu.VMEM((2, IN_DIM), jnp.float32),
            pltpu.SMEM((2,), jnp.float32),
            pltpu.SMEM((2,), jnp.float32),
        ],
    )(x, nt16, cs16, gw08, Wq0)


def _tc2_body(al_ref, ar_ref, nt_ref, cd_ref, w0_ref, b0_ref, gw_ref,
              wq_ref, o0_ref, bsum_ref, out_ref, acc_ref, m_ref, s_ref):
    i = pl.program_id(0)
    nd = lax.rsqrt(cd_ref[:, 0:1])
    al = al_ref[...] * nd
    ar = ar_ref[...] * nd
    z = (jnp.dot(al, w0_ref[:HALF, :], preferred_element_type=jnp.float32)
         + jnp.dot(ar, w0_ref[HALF:, :], preferred_element_type=jnp.float32))
    h = jnp.maximum(z + b0_ref[...], 0.0)

    g8 = jnp.dot(h, gw_ref[...], preferred_element_type=jnp.float32)
    _online_pool_update(i, g8[:, 0:1], h, nt_ref[:, 0:1],
                        acc_ref, m_ref, s_ref)

    @pl.when(i == pl.num_programs(0) - 1)
    def _():
        pcat = _pool_finish(acc_ref, s_ref)
        o1 = jnp.dot(pcat, wq_ref[...], preferred_element_type=jnp.float32)
        out_ref[...] = (o0_ref[...] + o1 + bsum_ref[...]) * 0.25


def _tc2(aggl, aggr, nt16, cd16, W0, b0r, gw18, Wq1, o0, bsum):
    return pl.pallas_call(
        _tc2_body,
        grid=(NBLK,),
        in_specs=[
            pl.BlockSpec((BLK, HALF), lambda i: (i, 0)),
            pl.BlockSpec((BLK, HALF), lambda i: (i, 0)),
            pl.BlockSpec((BLK, 16), lambda i: (i, 0)),
            pl.BlockSpec((BLK, HALF), lambda i: (i, 0)),
            pl.BlockSpec((IN_DIM, HID), lambda i: (0, 0)),
            pl.BlockSpec((1, HID), lambda i: (0, 0)),
            pl.BlockSpec((HID, 8), lambda i: (0, 0)),
            pl.BlockSpec((2 * HID, OUT_DIM), lambda i: (0, 0)),
            pl.BlockSpec((1, OUT_DIM), lambda i: (0, 0)),
            pl.BlockSpec((1, OUT_DIM), lambda i: (0, 0)),
        ],
        out_specs=pl.BlockSpec((1, OUT_DIM), lambda i: (0, 0)),
        out_shape=jax.ShapeDtypeStruct((1, OUT_DIM), jnp.float32),
        scratch_shapes=[
            pltpu.VMEM((2, HID), jnp.float32),
            pltpu.SMEM((2,), jnp.float32),
            pltpu.SMEM((2,), jnp.float32),
        ],
    )(aggl, aggr, nt16, cd16, W0, b0r, gw18, Wq1, o0, bsum)


def kernel(x, edge_index, node_type, W0, b0, W1, b1, gateW0, gateb0,
           gateW1, gateb1, Wp00, bp00, Wp01, bp01, Wp10, bp10, Wp11, bp11):
    epad = E_PAD - E
    src = jnp.pad(edge_index[0].astype(jnp.int32), (0, epad),
                  constant_values=N_PAD - 1)
    dst = jnp.pad(edge_index[1].astype(jnp.int32), (0, epad),
                  constant_values=N_PAD - 1)
    pad = N_PAD - N
    xp = jnp.pad(x, ((0, pad), (0, 0)))
    ntp = jnp.pad(node_type.astype(jnp.int32), (0, pad), constant_values=2)
    nt16 = jnp.broadcast_to(ntp[:, None], (N_PAD, 16))

    cs16, cd16 = _bincount()(src, dst)

    gw08 = jnp.broadcast_to(gateW0, (IN_DIM, 8))
    Wq0 = jnp.concatenate([Wp00, Wp10], axis=0)
    xsl, xsr, o0 = _tc1(xp, nt16, cs16, gw08, Wq0)

    aggl, aggr = _aggregate()(xsl, xsr, src, dst)

    gw18 = jnp.broadcast_to(gateW1, (HID, 8))
    Wq1 = jnp.concatenate([Wp01, Wp11], axis=0)
    bsum = (bp00 + bp01 + bp10 + bp11).reshape(1, OUT_DIM)
    out = _tc2(aggl, aggr, nt16, cd16, W0, b0.reshape(1, HID), gw18, Wq1,
               o0, bsum)
    return out.reshape(OUT_DIM)


# speedup vs baseline: 4.5587x; 1.0629x over previous
import functools

import jax
import jax.numpy as jnp
from jax import lax
from jax.experimental import pallas as pl
from jax.experimental.pallas import tpu as pltpu
from jax.experimental.pallas import tpu_sc as plsc

N = 10000
N_PAD = 10240
E = 160000
IN_DIM = 256
HID = 512
OUT_DIM = 64
HALF = 128

NTILES = 16
CHUNK = 128
E_PAD = 163840
EPT = E_PAD // NTILES
NFULL = EPT // CHUNK
CPB = 16
NBLKE = NFULL // CPB
RPT = N_PAD // NTILES

BLK = 1024
NBLK = N_PAD // BLK

@functools.lru_cache(maxsize=None)
def _sc_mesh():
    return plsc.VectorSubcoreMesh(core_axis_name="c", subcore_axis_name="s")


def _bincount_body(src_hbm, dst_hbm, cs_hbm, cd_hbm,
                   idx_v, ones_v, cnt_sh, sem):
    c = lax.axis_index("c")
    s = lax.axis_index("s")

    ones16 = jnp.full((16,), 1.0, jnp.float32)

    def fill(i, _):
        def fcol(g, _):
            ones_v[i, pl.ds(g * 16, 16)] = ones16
            return 0
        lax.fori_loop(0, HALF // 16, fcol, 0)
        return 0
    lax.fori_loop(0, CHUNK, fill, 0)

    def run(e_hbm, out_hbm):
        def seed_body(k, _):
            pltpu.sync_copy(ones_v, cnt_sh.at[pl.ds(s * RPT + k * CHUNK, CHUNK)])
            return 0
        lax.fori_loop(0, RPT // CHUNK, seed_body, 0)
        plsc.subcore_barrier()

        def block_body(k, _):
            pltpu.sync_copy(e_hbm.at[s, pl.ds(k * CPB, CPB)], idx_v)

            def quad(q, _):
                def fire(jl, _):
                    pltpu.async_copy(ones_v, cnt_sh.at[idx_v.at[q * 4 + jl]],
                                     sem, add=True)
                    return 0
                lax.fori_loop(0, 4, fire, 0)

                def drain(jl, _):
                    pltpu.make_async_copy(ones_v,
                                          cnt_sh.at[idx_v.at[q * 4 + jl]],
                                          sem).wait()
                    return 0
                lax.fori_loop(0, 4, drain, 0)
                return 0
            lax.fori_loop(0, CPB // 4, quad, 0)
            return 0
        lax.fori_loop(0, NBLKE, block_body, 0)

        plsc.subcore_barrier()

        def out_body(k, _):
            r0 = s * RPT + k * CHUNK
            pltpu.sync_copy(cnt_sh.at[pl.ds(r0, CHUNK)], ones_v)
            pltpu.sync_copy(ones_v, out_hbm.at[pl.ds(r0, CHUNK)])
            return 0
        lax.fori_loop(0, RPT // CHUNK, out_body, 0)

    @pl.when(c == 0)
    def _():
        run(src_hbm, cs_hbm)

    @pl.when(c == 1)
    def _():
        run(dst_hbm, cd_hbm)


@functools.lru_cache(maxsize=None)
def _bincount():
    return functools.partial(
        pl.kernel,
        out_type=[
            jax.ShapeDtypeStruct((N_PAD, HALF), jnp.float32),
            jax.ShapeDtypeStruct((N_PAD, HALF), jnp.float32),
        ],
        mesh=_sc_mesh(),
        scratch_types=[
            pltpu.VMEM((CPB, CHUNK), jnp.int32),
            pltpu.VMEM((CHUNK, HALF), jnp.float32),
            pltpu.VMEM_SHARED((N_PAD, HALF), jnp.float32),
            pltpu.SemaphoreType.DMA,
        ],
    )(_bincount_body)


def _agg_body(xsl_hbm, xsr_hbm, src_hbm, dst_hbm, al_hbm, ar_hbm,
              gi_v, di_v, rows0_v, rows1_v, acc_sh, sem0, sem1):
    c = lax.axis_index("c")
    s = lax.axis_index("s")

    def run(xs_hbm, out_hbm):
        def seed_body(k, _):
            r0 = s * RPT + k * CHUNK
            pltpu.sync_copy(xs_hbm.at[pl.ds(r0, CHUNK)], rows0_v)
            pltpu.sync_copy(rows0_v, acc_sh.at[pl.ds(r0, CHUNK)])
            return 0
        lax.fori_loop(0, RPT // CHUNK, seed_body, 0)
        plsc.subcore_barrier()

        bufs = ((rows0_v, sem0), (rows1_v, sem1))

        def block_body(k, _):
            pltpu.sync_copy(src_hbm.at[s, pl.ds(k * CPB, CPB)], gi_v)
            pltpu.sync_copy(dst_hbm.at[s, pl.ds(k * CPB, CPB)], di_v)

            for b in range(2):
                rows, sem = bufs[b]
                pltpu.async_copy(xs_hbm.at[gi_v.at[b]], rows, sem)

            def chunk_body(j2, _):
                for b in range(2):
                    jj = j2 * 2 + b
                    rows, sem = bufs[b]
                    pltpu.make_async_copy(xs_hbm.at[gi_v.at[jj]], rows,
                                          sem).wait()
                    pltpu.sync_copy(rows, acc_sh.at[di_v.at[jj]], add=True)

                    @pl.when(jj + 2 < CPB)
                    def _():
                        pltpu.async_copy(xs_hbm.at[gi_v.at[jj + 2]], rows, sem)
                return 0
            lax.fori_loop(0, CPB // 2, chunk_body, 0)
            return 0
        lax.fori_loop(0, NBLKE, block_body, 0)

        plsc.subcore_barrier()

        def out_body(k, _):
            r0 = s * RPT + k * CHUNK
            pltpu.sync_copy(acc_sh.at[pl.ds(r0, CHUNK)], rows0_v)
            pltpu.sync_copy(rows0_v, out_hbm.at[pl.ds(r0, CHUNK)])
            return 0
        lax.fori_loop(0, RPT // CHUNK, out_body, 0)

    @pl.when(c == 0)
    def _():
        run(xsl_hbm, al_hbm)

    @pl.when(c == 1)
    def _():
        run(xsr_hbm, ar_hbm)


@functools.lru_cache(maxsize=None)
def _aggregate():
    return functools.partial(
        pl.kernel,
        out_type=[
            jax.ShapeDtypeStruct((N_PAD, HALF), jnp.float32),
            jax.ShapeDtypeStruct((N_PAD, HALF), jnp.float32),
        ],
        mesh=_sc_mesh(),
        scratch_types=[
            pltpu.VMEM((CPB, CHUNK), jnp.int32),
            pltpu.VMEM((CPB, CHUNK), jnp.int32),
            pltpu.VMEM((CHUNK, HALF), jnp.float32),
            pltpu.VMEM((CHUNK, HALF), jnp.float32),
            pltpu.VMEM_SHARED((N_PAD, HALF), jnp.float32),
            pltpu.SemaphoreType.DMA,
            pltpu.SemaphoreType.DMA,
        ],
    )(_agg_body)


def _online_pool_update(i, g, feats, nt_col, acc_ref, m_ref, s_ref):
    @pl.when(i == 0)
    def _():
        m_ref[0] = -jnp.inf
        m_ref[1] = -jnp.inf
        s_ref[0] = 0.0
        s_ref[1] = 0.0
        acc_ref[...] = jnp.zeros_like(acc_ref)

    for t in range(2):
        mask = nt_col == t
        gm = jnp.max(jnp.where(mask, g, -jnp.inf))
        m_old = m_ref[t]
        m_new = jnp.maximum(m_old, gm)
        scale = jnp.where(m_old == -jnp.inf, 0.0, jnp.exp(m_old - m_new))
        w = jnp.where(mask, jnp.exp(g - m_new), 0.0)
        s_ref[t] = s_ref[t] * scale + jnp.sum(w)
        contrib = lax.dot_general(
            w, feats, (((0,), (0,)), ((), ())),
            preferred_element_type=jnp.float32)
        acc_ref[t:t + 1, :] = acc_ref[t:t + 1, :] * scale + contrib
        m_ref[t] = m_new


def _pool_finish(acc_ref, s_ref):
    s0 = s_ref[0]
    s1 = s_ref[1]
    p0 = jnp.where(s0 > 0, acc_ref[0:1, :] / s0, 0.0)
    p1 = jnp.where(s1 > 0, acc_ref[1:2, :] / s1, 0.0)
    return jnp.concatenate([p0, p1], axis=1)


def _tc1_body(x_ref, nt_ref, cs_ref, gw_ref, wq_ref,
              xsl_ref, xsr_ref, o0_ref, acc_ref, m_ref, s_ref):
    i = pl.program_id(0)
    xb = x_ref[...]
    nsrc = lax.rsqrt(cs_ref[:, 0:1])
    xs = xb * nsrc
    xsl_ref[...] = xs[:, :HALF]
    xsr_ref[...] = xs[:, HALF:]

    g8 = jnp.dot(xb, gw_ref[...], preferred_element_type=jnp.float32)
    _online_pool_update(i, g8[:, 0:1], xb, nt_ref[:, 0:1],
                        acc_ref, m_ref, s_ref)

    @pl.when(i == pl.num_programs(0) - 1)
    def _():
        pcat = _pool_finish(acc_ref, s_ref)
        o0_ref[...] = jnp.dot(pcat, wq_ref[...],
                              preferred_element_type=jnp.float32)


def _tc1(x, nt16, cs16, gw08, Wq0):
    return pl.pallas_call(
        _tc1_body,
        grid=(NBLK,),
        in_specs=[
            pl.BlockSpec((BLK, IN_DIM), lambda i: (i, 0)),
            pl.BlockSpec((BLK, 16), lambda i: (i, 0)),
            pl.BlockSpec((BLK, HALF), lambda i: (i, 0)),
            pl.BlockSpec((IN_DIM, 8), lambda i: (0, 0)),
            pl.BlockSpec((2 * IN_DIM, OUT_DIM), lambda i: (0, 0)),
        ],
        out_specs=[
            pl.BlockSpec((BLK, HALF), lambda i: (i, 0)),
            pl.BlockSpec((BLK, HALF), lambda i: (i, 0)),
            pl.BlockSpec((1, OUT_DIM), lambda i: (0, 0)),
        ],
        out_shape=[
            jax.ShapeDtypeStruct((N_PAD, HALF), jnp.float32),
            jax.ShapeDtypeStruct((N_PAD, HALF), jnp.float32),
            jax.ShapeDtypeStruct((1, OUT_DIM), jnp.float32),
        ],
        scratch_shapes=[
            pltpu.VMEM((2, IN_DIM), jnp.float32),
            pltpu.SMEM((2,), jnp.float32),
            pltpu.SMEM((2,), jnp.float32),
        ],
    )(x, nt16, cs16, gw08, Wq0)


def _tc2_body(al_ref, ar_ref, nt_ref, cd_ref, w0_ref, b0_ref, gw_ref,
              wq_ref, o0_ref, bsum_ref, out_ref, acc_ref, m_ref, s_ref):
    i = pl.program_id(0)
    nd = lax.rsqrt(cd_ref[:, 0:1])
    al = al_ref[...] * nd
    ar = ar_ref[...] * nd
    z = (jnp.dot(al, w0_ref[:HALF, :], preferred_element_type=jnp.float32)
         + jnp.dot(ar, w0_ref[HALF:, :], preferred_element_type=jnp.float32))
    h = jnp.maximum(z + b0_ref[...], 0.0)

    g8 = jnp.dot(h, gw_ref[...], preferred_element_type=jnp.float32)
    _online_pool_update(i, g8[:, 0:1], h, nt_ref[:, 0:1],
                        acc_ref, m_ref, s_ref)

    @pl.when(i == pl.num_programs(0) - 1)
    def _():
        pcat = _pool_finish(acc_ref, s_ref)
        o1 = jnp.dot(pcat, wq_ref[...], preferred_element_type=jnp.float32)
        out_ref[...] = (o0_ref[...] + o1 + bsum_ref[...]) * 0.25


def _tc2(aggl, aggr, nt16, cd16, W0, b0r, gw18, Wq1, o0, bsum):
    return pl.pallas_call(
        _tc2_body,
        grid=(NBLK,),
        in_specs=[
            pl.BlockSpec((BLK, HALF), lambda i: (i, 0)),
            pl.BlockSpec((BLK, HALF), lambda i: (i, 0)),
            pl.BlockSpec((BLK, 16), lambda i: (i, 0)),
            pl.BlockSpec((BLK, HALF), lambda i: (i, 0)),
            pl.BlockSpec((IN_DIM, HID), lambda i: (0, 0)),
            pl.BlockSpec((1, HID), lambda i: (0, 0)),
            pl.BlockSpec((HID, 8), lambda i: (0, 0)),
            pl.BlockSpec((2 * HID, OUT_DIM), lambda i: (0, 0)),
            pl.BlockSpec((1, OUT_DIM), lambda i: (0, 0)),
            pl.BlockSpec((1, OUT_DIM), lambda i: (0, 0)),
        ],
        out_specs=pl.BlockSpec((1, OUT_DIM), lambda i: (0, 0)),
        out_shape=jax.ShapeDtypeStruct((1, OUT_DIM), jnp.float32),
        scratch_shapes=[
            pltpu.VMEM((2, HID), jnp.float32),
            pltpu.SMEM((2,), jnp.float32),
            pltpu.SMEM((2,), jnp.float32),
        ],
    )(aggl, aggr, nt16, cd16, W0, b0r, gw18, Wq1, o0, bsum)


def kernel(x, edge_index, node_type, W0, b0, W1, b1, gateW0, gateb0,
           gateW1, gateb1, Wp00, bp00, Wp01, bp01, Wp10, bp10, Wp11, bp11):
    epad = E_PAD - E
    src = jnp.pad(edge_index[0].astype(jnp.int32), (0, epad),
                  constant_values=N_PAD - 1).reshape(NTILES, NFULL, CHUNK)
    dst = jnp.pad(edge_index[1].astype(jnp.int32), (0, epad),
                  constant_values=N_PAD - 1).reshape(NTILES, NFULL, CHUNK)
    pad = N_PAD - N
    xp = jnp.pad(x, ((0, pad), (0, 0)))
    ntp = jnp.pad(node_type.astype(jnp.int32), (0, pad), constant_values=2)
    nt16 = jnp.broadcast_to(ntp[:, None], (N_PAD, 16))

    cs16, cd16 = _bincount()(src, dst)

    gw08 = jnp.broadcast_to(gateW0, (IN_DIM, 8))
    Wq0 = jnp.concatenate([Wp00, Wp10], axis=0)
    xsl, xsr, o0 = _tc1(xp, nt16, cs16, gw08, Wq0)

    aggl, aggr = _aggregate()(xsl, xsr, src, dst)

    gw18 = jnp.broadcast_to(gateW1, (HID, 8))
    Wq1 = jnp.concatenate([Wp01, Wp11], axis=0)
    bsum = (bp00 + bp01 + bp10 + bp11).reshape(1, OUT_DIM)
    out = _tc2(aggl, aggr, nt16, cd16, W0, b0.reshape(1, HID), gw18, Wq1,
               o0, bsum)
    return out.reshape(OUT_DIM)
